# baseline (speedup 1.0000x reference)
import functools

import jax
import jax.numpy as jnp
from jax import lax
from jax.experimental import pallas as pl
from jax.experimental.pallas import tpu as pltpu
from jax.experimental.pallas import tpu_sc as plsc

B, T, C = 4, 8192, 1024
NC, NS = 2, 16
NW = NC * NS
T_W = T // NW
R = 8
NCHUNK = T_W // R
NC2 = NCHUNK // 2
LANES = 16
NVEC = C // LANES

_mesh = plsc.VectorSubcoreMesh(core_axis_name="c", subcore_axis_name="s")


@functools.partial(
    pl.kernel,
    out_type=jax.ShapeDtypeStruct((B, T, C), jnp.float32),
    mesh=_mesh,
    scratch_types=(
        [pltpu.VMEM((R, C), jnp.float32)] * 8
        + [pltpu.VMEM((R, C), jnp.float32)] * 2
        + [pltpu.SemaphoreType.DMA] * 8
        + [pltpu.SemaphoreType.DMA] * 8
        + [pltpu.SemaphoreType.DMA] * 2
    ),
)
def _pe_add_sc(x_hbm, pe_hbm, out_hbm, *refs):
    xbufs = refs[0:8]
    pbufs = refs[8:10]
    sin = refs[10:18]
    sout = refs[18:26]
    spe = refs[26:28]

    wid = lax.axis_index("s") * NC + lax.axis_index("c")
    tw0 = wid * T_W

    def t0_of(chunk):
        return tw0 + chunk * R

    def start_in(b, chunk, u):
        pltpu.async_copy(x_hbm.at[b, pl.ds(t0_of(chunk), R)], xbufs[u],
                         sin[u])

    def wait_in(b, chunk, u):
        pltpu.make_async_copy(
            x_hbm.at[b, pl.ds(t0_of(chunk), R)], xbufs[u], sin[u]).wait()

    def start_out(b, chunk, u):
        pltpu.async_copy(xbufs[u], out_hbm.at[b, pl.ds(t0_of(chunk), R)],
                         sout[u])

    def wait_out(b, chunk, u):
        pltpu.make_async_copy(
            xbufs[u], out_hbm.at[b, pl.ds(t0_of(chunk), R)], sout[u]).wait()

    def start_pe(chunk, q):
        pltpu.async_copy(pe_hbm.at[pl.ds(t0_of(chunk), R)], pbufs[q], spe[q])

    def wait_pe(chunk, q):
        pltpu.make_async_copy(
            pe_hbm.at[pl.ds(t0_of(chunk), R)], pbufs[q], spe[q]).wait()

    NSEG = 16

    def add_pe_fused(cc, q):
        bufs = [xbufs[cc * 4 + b] for b in range(B)]
        pb = pbufs[q]

        def row(r, carry):
            for seg in range(NVEC // NSEG):
                pe_vecs = []
                for j in range(NSEG):
                    sl = pl.ds((seg * NSEG + j) * LANES, LANES)
                    pe_vecs.append(pb[r, sl])
                for b in range(B):
                    xb = bufs[b]
                    for j in range(NSEG):
                        sl = pl.ds((seg * NSEG + j) * LANES, LANES)
                        xb[r, sl] = xb[r, sl] + pe_vecs[j]
            return carry

        lax.fori_loop(0, R, row, 0)

    start_pe(0, 0)
    for b in range(B):
        start_in(b, 0, b)

    def outer(c2, carry):
        for cc in range(2):
            chunk = c2 * 2 + cc
            q = cc
            if cc == 0:
                start_pe(chunk + 1, 1)
            else:
                @pl.when(c2 < NC2 - 1)
                def _():
                    start_pe(chunk + 1, 0)
            wait_pe(chunk, q)
            for b in range(B):
                wait_in(b, chunk, cc * 4 + b)
            add_pe_fused(cc, q)
            for b in range(B):
                start_out(b, chunk, cc * 4 + b)
            for b in range(B):
                v = (1 - cc) * 4 + b
                if cc == 1:
                    @pl.when(c2 < NC2 - 1)
                    def _():
                        wait_out(b, chunk - 1, v)
                        start_in(b, chunk + 1, v)
                else:
                    @pl.when(c2 >= 1)
                    def _():
                        wait_out(b, chunk - 1, v)
                    start_in(b, chunk + 1, v)
        return carry

    lax.fori_loop(0, NC2, outer, 0)

    for b in range(B):
        wait_out(b, NCHUNK - 2, b)
    for b in range(B):
        wait_out(b, NCHUNK - 1, 4 + b)


def kernel(x, pos_emb):
    return _pe_add_sc(x, pos_emb)


# speedup vs baseline: 1.4525x; 1.4525x over previous
import functools

import jax
import jax.numpy as jnp
from jax import lax
from jax.experimental import pallas as pl
from jax.experimental.pallas import tpu as pltpu
from jax.experimental.pallas import tpu_sc as plsc

B, T, C = 4, 8192, 1024
NC, NS = 2, 16
NW = NC * NS
T_W = T // NW
R = 8
NCHUNK = T_W // R
NC2 = NCHUNK // 2
LANES = 16
NVEC = C // LANES

_mesh = plsc.VectorSubcoreMesh(core_axis_name="c", subcore_axis_name="s")


@functools.partial(
    pl.kernel,
    out_type=jax.ShapeDtypeStruct((B, T, C), jnp.float32),
    mesh=_mesh,
    scratch_types=(
        [pltpu.VMEM((R, C), jnp.float32)] * 8
        + [pltpu.VMEM((R, C), jnp.float32)] * 2
        + [pltpu.SemaphoreType.DMA] * 8
        + [pltpu.SemaphoreType.DMA] * 8
        + [pltpu.SemaphoreType.DMA] * 2
    ),
)
def _pe_add_sc(x_hbm, pe_hbm, out_hbm, *refs):
    xbufs = refs[0:8]
    pbufs = refs[8:10]
    sin = refs[10:18]
    sout = refs[18:26]
    spe = refs[26:28]

    wid = lax.axis_index("s") * NC + lax.axis_index("c")
    tw0 = wid * T_W

    def t0_of(chunk):
        return tw0 + chunk * R

    def start_in(b, chunk, u):
        pltpu.async_copy(x_hbm.at[b, pl.ds(t0_of(chunk), R)], xbufs[u],
                         sin[u])

    def wait_in(b, chunk, u):
        pltpu.make_async_copy(
            x_hbm.at[b, pl.ds(t0_of(chunk), R)], xbufs[u], sin[u]).wait()

    def start_out(b, chunk, u):
        pltpu.async_copy(xbufs[u], out_hbm.at[b, pl.ds(t0_of(chunk), R)],
                         sout[u])

    def wait_out(b, chunk, u):
        pltpu.make_async_copy(
            xbufs[u], out_hbm.at[b, pl.ds(t0_of(chunk), R)], sout[u]).wait()

    def start_pe(chunk, q):
        pltpu.async_copy(pe_hbm.at[pl.ds(t0_of(chunk), R)], pbufs[q], spe[q])

    def wait_pe(chunk, q):
        pltpu.make_async_copy(
            pe_hbm.at[pl.ds(t0_of(chunk), R)], pbufs[q], spe[q]).wait()

    def add_pe(u, q):
        xb, pb = xbufs[u], pbufs[q]

        def row(r, carry):
            for j in range(NVEC):
                sl = pl.ds(j * LANES, LANES)
                xb[r, sl] = xb[r, sl] + pb[r, sl]
            return carry

        lax.fori_loop(0, R, row, 0)

    start_pe(0, 0)
    for b in range(B):
        start_in(b, 0, b)

    def outer(c2, carry):
        for cc in range(2):
            chunk = c2 * 2 + cc
            q = cc
            if cc == 0:
                start_pe(chunk + 1, 1)
            else:
                @pl.when(c2 < NC2 - 1)
                def _():
                    start_pe(chunk + 1, 0)
            wait_pe(chunk, q)
            for b in range(B):
                u = cc * 4 + b
                v = (1 - cc) * 4 + b
                wait_in(b, chunk, u)
                add_pe(u, q)
                start_out(b, chunk, u)
                if cc == 1:
                    @pl.when(c2 < NC2 - 1)
                    def _():
                        wait_out(b, chunk - 1, v)
                        start_in(b, chunk + 1, v)
                else:
                    @pl.when(c2 >= 1)
                    def _():
                        wait_out(b, chunk - 1, v)
                    start_in(b, chunk + 1, v)
        return carry

    lax.fori_loop(0, NC2, outer, 0)

    for b in range(B):
        wait_out(b, NCHUNK - 2, b)
    for b in range(B):
        wait_out(b, NCHUNK - 1, 4 + b)


def kernel(x, pos_emb):
    return _pe_add_sc(x, pos_emb)
